# baseline (speedup 1.0000x reference)
import jax
import jax.numpy as jnp
from jax.experimental import pallas as pl
from jax.experimental.pallas import tpu as pltpu

_EPS = 1e-5
_ROWS = 256


def _eln_kernel(x0_ref, x1_ref, x2_ref, w_ref, o0_ref, o1_ref, o2_ref):
    d = x0_ref.shape[-1]
    inv_d = 1.0 / d

    x0 = x0_ref[0]
    x1 = x1_ref[0]
    x2 = x2_ref[0]

    xc0 = x0 - jnp.sum(x0, axis=-1, keepdims=True) * inv_d
    xc1 = x1 - jnp.sum(x1, axis=-1, keepdims=True) * inv_d
    xc2 = x2 - jnp.sum(x2, axis=-1, keepdims=True) * inv_d

    b00 = jnp.sum(xc0 * xc0, axis=-1, keepdims=True) * inv_d + 2.0 * _EPS
    b11 = jnp.sum(xc1 * xc1, axis=-1, keepdims=True) * inv_d + 3.0 * _EPS
    b22 = jnp.sum(xc2 * xc2, axis=-1, keepdims=True) * inv_d + 4.0 * _EPS
    b01 = jnp.sum(xc0 * xc1, axis=-1, keepdims=True) * inv_d
    b02 = jnp.sum(xc0 * xc2, axis=-1, keepdims=True) * inv_d
    b12 = jnp.sum(xc1 * xc2, axis=-1, keepdims=True) * inv_d

    q = (b00 + b11 + b22) * (1.0 / 3.0)
    d0 = b00 - q
    d1 = b11 - q
    d2 = b22 - q
    p2 = d0 * d0 + d1 * d1 + d2 * d2 + 2.0 * (b01 * b01 + b02 * b02 + b12 * b12)
    p = jnp.sqrt(p2 * (1.0 / 6.0))
    det = (d0 * (d1 * d2 - b12 * b12)
           - b01 * (b01 * d2 - b12 * b02)
           + b02 * (b01 * b12 - d1 * b02))
    p3 = jnp.maximum(p * p * p, 1e-38)
    r = jnp.clip(0.5 * det / p3, -1.0, 1.0)
    ar = jnp.abs(r)
    small = ar < 0.5
    zz = jnp.where(small, r * r, 0.5 * (1.0 - ar))
    ss = jnp.where(small, ar, jnp.sqrt(zz))
    poly = zz * (1.6666586697e-01
                 + zz * (-4.2743422091e-02 + zz * (-8.6563630030e-03)))
    rz = poly / (1.0 + zz * (-7.0662963390e-01))
    t = ss + ss * rz
    acos_abs = jnp.where(small, (jnp.pi / 2.0) - t, 2.0 * t)
    acos_r = jnp.where(r >= 0.0, acos_abs, jnp.pi - acos_abs)
    phi = acos_r * (1.0 / 3.0)
    u = phi * phi
    cphi = 1.0 + u * (-0.5 + u * ((1.0 / 24.0)
                                  + u * (-(1.0 / 720.0) + u * (1.0 / 40320.0))))
    sphi = phi * (1.0 + u * (-(1.0 / 6.0)
                             + u * ((1.0 / 120.0)
                                    + u * (-(1.0 / 5040.0) + u * (1.0 / 362880.0)))))
    l3 = q + 2.0 * p * cphi
    l1 = q - p * cphi - jnp.float32(1.7320508075688772) * p * sphi
    l2 = 3.0 * q - l3 - l1
    floor = jnp.float32(1e-9)
    s1 = jnp.sqrt(jnp.maximum(l1, floor))
    s2 = jnp.sqrt(jnp.maximum(l2, floor))
    s3 = jnp.sqrt(jnp.maximum(l3, floor))

    c0 = 1.0 / s1
    c1 = -1.0 / (s1 * s2 * (s1 + s2))
    c2 = (s1 + s2 + s3) / ((s1 * s2 * s3) * ((s1 + s2) * (s2 + s3) * (s3 + s1)))

    sq00 = b00 * b00 + b01 * b01 + b02 * b02
    sq11 = b01 * b01 + b11 * b11 + b12 * b12
    sq22 = b02 * b02 + b12 * b12 + b22 * b22
    sq01 = b00 * b01 + b01 * b11 + b02 * b12
    sq02 = b00 * b02 + b01 * b12 + b02 * b22
    sq12 = b01 * b02 + b11 * b12 + b12 * b22
    lsum = l1 + l2
    lprod = l1 * l2
    m00 = c0 + c1 * (b00 - l1) + c2 * (sq00 - lsum * b00 + lprod)
    m11 = c0 + c1 * (b11 - l1) + c2 * (sq11 - lsum * b11 + lprod)
    m22 = c0 + c1 * (b22 - l1) + c2 * (sq22 - lsum * b22 + lprod)
    m01 = c1 * b01 + c2 * (sq01 - lsum * b01)
    m02 = c1 * b02 + c2 * (sq02 - lsum * b02)
    m12 = c1 * b12 + c2 * (sq12 - lsum * b12)

    w = w_ref[:, :]
    o0_ref[...] = (m00 * xc0 + m01 * xc1 + m02 * xc2) * w
    o1_ref[...] = (m01 * xc0 + m11 * xc1 + m12 * xc2) * w
    o2_ref[...] = (m02 * xc0 + m12 * xc1 + m22 * xc2) * w


@jax.jit
def kernel(x, weight):
    n, v, d = x.shape
    xt = jnp.swapaxes(x, 0, 1)
    w2 = weight.reshape(1, d)
    outs = pl.pallas_call(
        _eln_kernel,
        grid=(n // _ROWS,),
        in_specs=[
            pl.BlockSpec((1, _ROWS, d), lambda i: (0, i, 0)),
            pl.BlockSpec((1, _ROWS, d), lambda i: (1, i, 0)),
            pl.BlockSpec((1, _ROWS, d), lambda i: (2, i, 0)),
            pl.BlockSpec((1, d), lambda i: (0, 0)),
        ],
        out_specs=[
            pl.BlockSpec((_ROWS, d), lambda i: (i, 0)),
            pl.BlockSpec((_ROWS, d), lambda i: (i, 0)),
            pl.BlockSpec((_ROWS, d), lambda i: (i, 0)),
        ],
        out_shape=[jax.ShapeDtypeStruct((n, d), x.dtype)] * 3,
        compiler_params=pltpu.CompilerParams(
            dimension_semantics=("arbitrary",),
        ),
    )(xt, xt, xt, w2)
    return jnp.stack(outs, axis=1)


# speedup vs baseline: 1.0684x; 1.0684x over previous
import jax
import jax.numpy as jnp
from jax.experimental import pallas as pl
from jax.experimental.pallas import tpu as pltpu

_EPS = 1e-5
_ROWS = 512


def _eln_kernel(x0_ref, x1_ref, x2_ref, w_ref, o0_ref, o1_ref, o2_ref):
    d = x0_ref.shape[-1]
    inv_d = 1.0 / d

    x0 = x0_ref[0]
    x1 = x1_ref[0]
    x2 = x2_ref[0]

    xc0 = x0 - jnp.sum(x0, axis=-1, keepdims=True) * inv_d
    xc1 = x1 - jnp.sum(x1, axis=-1, keepdims=True) * inv_d
    xc2 = x2 - jnp.sum(x2, axis=-1, keepdims=True) * inv_d

    b00 = jnp.sum(xc0 * xc0, axis=-1, keepdims=True) * inv_d + 2.0 * _EPS
    b11 = jnp.sum(xc1 * xc1, axis=-1, keepdims=True) * inv_d + 3.0 * _EPS
    b22 = jnp.sum(xc2 * xc2, axis=-1, keepdims=True) * inv_d + 4.0 * _EPS
    b01 = jnp.sum(xc0 * xc1, axis=-1, keepdims=True) * inv_d
    b02 = jnp.sum(xc0 * xc2, axis=-1, keepdims=True) * inv_d
    b12 = jnp.sum(xc1 * xc2, axis=-1, keepdims=True) * inv_d

    q = (b00 + b11 + b22) * (1.0 / 3.0)
    d0 = b00 - q
    d1 = b11 - q
    d2 = b22 - q
    p2 = d0 * d0 + d1 * d1 + d2 * d2 + 2.0 * (b01 * b01 + b02 * b02 + b12 * b12)
    p = jnp.sqrt(p2 * (1.0 / 6.0))
    det = (d0 * (d1 * d2 - b12 * b12)
           - b01 * (b01 * d2 - b12 * b02)
           + b02 * (b01 * b12 - d1 * b02))
    p3 = jnp.maximum(p * p * p, 1e-38)
    r = jnp.clip(0.5 * det / p3, -1.0, 1.0)
    ar = jnp.abs(r)
    small = ar < 0.5
    zz = jnp.where(small, r * r, 0.5 * (1.0 - ar))
    ss = jnp.where(small, ar, jnp.sqrt(zz))
    poly = zz * (1.6666586697e-01
                 + zz * (-4.2743422091e-02 + zz * (-8.6563630030e-03)))
    rz = poly / (1.0 + zz * (-7.0662963390e-01))
    t = ss + ss * rz
    acos_abs = jnp.where(small, (jnp.pi / 2.0) - t, 2.0 * t)
    acos_r = jnp.where(r >= 0.0, acos_abs, jnp.pi - acos_abs)
    phi = acos_r * (1.0 / 3.0)
    u = phi * phi
    cphi = 1.0 + u * (-0.5 + u * ((1.0 / 24.0)
                                  + u * (-(1.0 / 720.0) + u * (1.0 / 40320.0))))
    sphi = phi * (1.0 + u * (-(1.0 / 6.0)
                             + u * ((1.0 / 120.0)
                                    + u * (-(1.0 / 5040.0) + u * (1.0 / 362880.0)))))
    l3 = q + 2.0 * p * cphi
    l1 = q - p * cphi - jnp.float32(1.7320508075688772) * p * sphi
    l2 = 3.0 * q - l3 - l1
    floor = jnp.float32(1e-9)
    s1 = jnp.sqrt(jnp.maximum(l1, floor))
    s2 = jnp.sqrt(jnp.maximum(l2, floor))
    s3 = jnp.sqrt(jnp.maximum(l3, floor))

    c0 = 1.0 / s1
    c1 = -1.0 / (s1 * s2 * (s1 + s2))
    c2 = (s1 + s2 + s3) / ((s1 * s2 * s3) * ((s1 + s2) * (s2 + s3) * (s3 + s1)))

    sq00 = b00 * b00 + b01 * b01 + b02 * b02
    sq11 = b01 * b01 + b11 * b11 + b12 * b12
    sq22 = b02 * b02 + b12 * b12 + b22 * b22
    sq01 = b00 * b01 + b01 * b11 + b02 * b12
    sq02 = b00 * b02 + b01 * b12 + b02 * b22
    sq12 = b01 * b02 + b11 * b12 + b12 * b22
    lsum = l1 + l2
    lprod = l1 * l2
    m00 = c0 + c1 * (b00 - l1) + c2 * (sq00 - lsum * b00 + lprod)
    m11 = c0 + c1 * (b11 - l1) + c2 * (sq11 - lsum * b11 + lprod)
    m22 = c0 + c1 * (b22 - l1) + c2 * (sq22 - lsum * b22 + lprod)
    m01 = c1 * b01 + c2 * (sq01 - lsum * b01)
    m02 = c1 * b02 + c2 * (sq02 - lsum * b02)
    m12 = c1 * b12 + c2 * (sq12 - lsum * b12)

    w = w_ref[:, :]
    o0_ref[...] = (m00 * xc0 + m01 * xc1 + m02 * xc2) * w
    o1_ref[...] = (m01 * xc0 + m11 * xc1 + m12 * xc2) * w
    o2_ref[...] = (m02 * xc0 + m12 * xc1 + m22 * xc2) * w


@jax.jit
def kernel(x, weight):
    n, v, d = x.shape
    xt = jnp.swapaxes(x, 0, 1)
    w2 = weight.reshape(1, d)
    outs = pl.pallas_call(
        _eln_kernel,
        grid=(n // _ROWS,),
        in_specs=[
            pl.BlockSpec((1, _ROWS, d), lambda i: (0, i, 0)),
            pl.BlockSpec((1, _ROWS, d), lambda i: (1, i, 0)),
            pl.BlockSpec((1, _ROWS, d), lambda i: (2, i, 0)),
            pl.BlockSpec((1, d), lambda i: (0, 0)),
        ],
        out_specs=[
            pl.BlockSpec((_ROWS, d), lambda i: (i, 0)),
            pl.BlockSpec((_ROWS, d), lambda i: (i, 0)),
            pl.BlockSpec((_ROWS, d), lambda i: (i, 0)),
        ],
        out_shape=[jax.ShapeDtypeStruct((n, d), x.dtype)] * 3,
        compiler_params=pltpu.CompilerParams(
            dimension_semantics=("arbitrary",),
        ),
    )(xt, xt, xt, w2)
    return jnp.stack(outs, axis=1)


# speedup vs baseline: 1.1713x; 1.0963x over previous
import jax
import jax.numpy as jnp
from jax.experimental import pallas as pl
from jax.experimental.pallas import tpu as pltpu

_EPS = 1e-5
_ROWS = 512


def _eln_kernel(x0_ref, x1_ref, x2_ref, w_ref, o0_ref, o1_ref, o2_ref):
    d = x0_ref.shape[-1]
    inv_d = 1.0 / d

    x0 = x0_ref[0]
    x1 = x1_ref[0]
    x2 = x2_ref[0]

    xc0 = x0 - jnp.sum(x0, axis=-1, keepdims=True) * inv_d
    xc1 = x1 - jnp.sum(x1, axis=-1, keepdims=True) * inv_d
    xc2 = x2 - jnp.sum(x2, axis=-1, keepdims=True) * inv_d

    b00 = jnp.sum(xc0 * xc0, axis=-1, keepdims=True) * inv_d + 2.0 * _EPS
    b11 = jnp.sum(xc1 * xc1, axis=-1, keepdims=True) * inv_d + 3.0 * _EPS
    b22 = jnp.sum(xc2 * xc2, axis=-1, keepdims=True) * inv_d + 4.0 * _EPS
    b01 = jnp.sum(xc0 * xc1, axis=-1, keepdims=True) * inv_d
    b02 = jnp.sum(xc0 * xc2, axis=-1, keepdims=True) * inv_d
    b12 = jnp.sum(xc1 * xc2, axis=-1, keepdims=True) * inv_d

    bT = jnp.transpose(jnp.concatenate(
        [b00, b11, b22, b01, b02, b12, b00, b00], axis=1))
    b00_, b11_, b22_ = bT[0:1], bT[1:2], bT[2:3]
    b01_, b02_, b12_ = bT[3:4], bT[4:5], bT[5:6]

    q = (b00_ + b11_ + b22_) * (1.0 / 3.0)
    d0 = b00_ - q
    d1 = b11_ - q
    d2 = b22_ - q
    p2 = d0 * d0 + d1 * d1 + d2 * d2 + 2.0 * (b01_ * b01_ + b02_ * b02_ + b12_ * b12_)
    p = jnp.sqrt(p2 * (1.0 / 6.0))
    det = (d0 * (d1 * d2 - b12_ * b12_)
           - b01_ * (b01_ * d2 - b12_ * b02_)
           + b02_ * (b01_ * b12_ - d1 * b02_))
    p3 = jnp.maximum(p * p * p, 1e-38)
    r = jnp.clip(0.5 * det / p3, -1.0, 1.0)
    ar = jnp.abs(r)
    small = ar < 0.5
    zz = jnp.where(small, r * r, 0.5 * (1.0 - ar))
    ss = jnp.where(small, ar, jnp.sqrt(zz))
    poly = zz * (1.6666586697e-01
                 + zz * (-4.2743422091e-02 + zz * (-8.6563630030e-03)))
    rz = poly / (1.0 + zz * (-7.0662963390e-01))
    t = ss + ss * rz
    acos_abs = jnp.where(small, (jnp.pi / 2.0) - t, 2.0 * t)
    acos_r = jnp.where(r >= 0.0, acos_abs, jnp.pi - acos_abs)
    phi = acos_r * (1.0 / 3.0)
    u = phi * phi
    cphi = 1.0 + u * (-0.5 + u * ((1.0 / 24.0)
                                  + u * (-(1.0 / 720.0) + u * (1.0 / 40320.0))))
    sphi = phi * (1.0 + u * (-(1.0 / 6.0)
                             + u * ((1.0 / 120.0)
                                    + u * (-(1.0 / 5040.0) + u * (1.0 / 362880.0)))))
    l3 = q + 2.0 * p * cphi
    l1 = q - p * cphi - jnp.float32(1.7320508075688772) * p * sphi
    l2 = 3.0 * q - l3 - l1
    floor = jnp.float32(1e-9)
    s1 = jnp.sqrt(jnp.maximum(l1, floor))
    s2 = jnp.sqrt(jnp.maximum(l2, floor))
    s3 = jnp.sqrt(jnp.maximum(l3, floor))

    c0 = 1.0 / s1
    c1 = -1.0 / (s1 * s2 * (s1 + s2))
    c2 = (s1 + s2 + s3) / ((s1 * s2 * s3) * ((s1 + s2) * (s2 + s3) * (s3 + s1)))

    sq00 = b00_ * b00_ + b01_ * b01_ + b02_ * b02_
    sq11 = b01_ * b01_ + b11_ * b11_ + b12_ * b12_
    sq22 = b02_ * b02_ + b12_ * b12_ + b22_ * b22_
    sq01 = b00_ * b01_ + b01_ * b11_ + b02_ * b12_
    sq02 = b00_ * b02_ + b01_ * b12_ + b02_ * b22_
    sq12 = b01_ * b02_ + b11_ * b12_ + b12_ * b22_
    lsum = l1 + l2
    lprod = l1 * l2
    m00_ = c0 + c1 * (b00_ - l1) + c2 * (sq00 - lsum * b00_ + lprod)
    m11_ = c0 + c1 * (b11_ - l1) + c2 * (sq11 - lsum * b11_ + lprod)
    m22_ = c0 + c1 * (b22_ - l1) + c2 * (sq22 - lsum * b22_ + lprod)
    m01_ = c1 * b01_ + c2 * (sq01 - lsum * b01_)
    m02_ = c1 * b02_ + c2 * (sq02 - lsum * b02_)
    m12_ = c1 * b12_ + c2 * (sq12 - lsum * b12_)


    mT = jnp.transpose(jnp.concatenate(
        [m00_, m11_, m22_, m01_, m02_, m12_, m00_, m00_], axis=0))
    m00, m11, m22 = mT[:, 0:1], mT[:, 1:2], mT[:, 2:3]
    m01, m02, m12 = mT[:, 3:4], mT[:, 4:5], mT[:, 5:6]

    w = w_ref[:, :]
    o0_ref[...] = (m00 * xc0 + m01 * xc1 + m02 * xc2) * w
    o1_ref[...] = (m01 * xc0 + m11 * xc1 + m12 * xc2) * w
    o2_ref[...] = (m02 * xc0 + m12 * xc1 + m22 * xc2) * w


@jax.jit
def kernel(x, weight):
    n, v, d = x.shape
    xt = jnp.swapaxes(x, 0, 1)
    w2 = weight.reshape(1, d)
    outs = pl.pallas_call(
        _eln_kernel,
        grid=(n // _ROWS,),
        in_specs=[
            pl.BlockSpec((1, _ROWS, d), lambda i: (0, i, 0)),
            pl.BlockSpec((1, _ROWS, d), lambda i: (1, i, 0)),
            pl.BlockSpec((1, _ROWS, d), lambda i: (2, i, 0)),
            pl.BlockSpec((1, d), lambda i: (0, 0)),
        ],
        out_specs=[
            pl.BlockSpec((_ROWS, d), lambda i: (i, 0)),
            pl.BlockSpec((_ROWS, d), lambda i: (i, 0)),
            pl.BlockSpec((_ROWS, d), lambda i: (i, 0)),
        ],
        out_shape=[jax.ShapeDtypeStruct((n, d), x.dtype)] * 3,
        compiler_params=pltpu.CompilerParams(
            dimension_semantics=("arbitrary",),
        ),
    )(xt, xt, xt, w2)
    return jnp.stack(outs, axis=1)
